# baseline (speedup 1.0000x reference)
import jax
import jax.numpy as jnp
from jax import lax
from jax.experimental import pallas as pl
from jax.experimental.pallas import tpu as pltpu
from jax.experimental.pallas import tpu_sc as plsc

B = 16384
DIM = 16
EPS = 1e-6

_NC = 2
_NS = 16
_NW = _NC * _NS
_BPW = B // _NW
_G = 16
_NG = _BPW // _G
_CH = 128
_R = B // _CH


def _sc_body(idx1_hbm, idx2_hbm, t1_hbm, t2_hbm, out_hbm,
             idx1_v, idx2_v, buf1_v, buf2_v,
             out_v, sem, sem2):
    c = lax.axis_index("c")
    s = lax.axis_index("s")
    wid = s * _NC + c
    base = wid * _BPW

    cp3 = pltpu.async_copy(idx1_hbm.at[pl.ds(base, _BPW)], idx1_v, sem2)
    cp4 = pltpu.async_copy(idx2_hbm.at[pl.ds(base, _BPW)], idx2_v, sem2)
    cp3.wait()
    cp4.wait()

    lanes = lax.iota(jnp.int32, 16)

    @pl.loop(0, _NG)
    def _group(g):
        r0 = g * _G
        sl = pl.ds(r0, _G)
        iv1 = idx1_v[sl]
        iv2 = idx2_v[sl]
        cv1 = (iv1 >> 7) * 128
        cv2 = (iv2 >> 7) * 128
        copies = []
        for l in range(_G):
            c1 = pl.multiple_of(cv1[l], 128)
            c2 = pl.multiple_of(cv2[l], 128)
            copies.append(pltpu.async_copy(
                t1_hbm.at[:, pl.ds(c1, 128)], buf1_v.at[l], sem))
            copies.append(pltpu.async_copy(
                t2_hbm.at[:, pl.ds(c2, 128)], buf2_v.at[l], sem))
        for cp in copies:
            cp.wait()

        sub1 = iv1 & 127
        sub2 = iv2 & 127
        dot = jnp.zeros((16,), jnp.float32)
        s1 = jnp.zeros((16,), jnp.float32)
        s2 = jnp.zeros((16,), jnp.float32)
        for d in range(DIM):
            dv = jnp.full((16,), d, jnp.int32)
            v1 = plsc.load_gather(buf1_v, [lanes, dv, sub1])
            v2 = plsc.load_gather(buf2_v, [lanes, dv, sub2])
            dot = dot + v1 * v2
            s1 = s1 + v1 * v1
            s2 = s2 + v2 * v2
        out_v[sl] = _logsigmoid_cos(dot, s1, s2)

    pltpu.sync_copy(out_v, out_hbm.at[pl.ds(base, _BPW)])


def _logsigmoid_cos(dot, s1, s2):
    p = s1 * s2
    i = plsc.bitcast(p, jnp.int32)
    y = plsc.bitcast(jnp.int32(0x5F3759DF) - (i >> 1), jnp.float32)
    for _ in range(3):
        y = y * (1.5 - 0.5 * p * y * y)
    den = jnp.maximum(p * y, EPS)
    cos = dot / den
    x = 100.0 * cos
    u = jnp.exp(-jnp.abs(x))
    yy = 1.0 + u
    bits = plsc.bitcast(yy, jnp.int32).astype(jnp.float32)
    z = (bits * (2.0 ** -23) - 127.0) * 0.6931471805599453
    for _ in range(3):
        z = z - 1.0 + yy * jnp.exp(-z)
    return jnp.minimum(x, 0.0) - z


_sc_reduce = pl.kernel(
    _sc_body,
    out_type=jax.ShapeDtypeStruct((B,), jnp.float32),
    mesh=plsc.VectorSubcoreMesh(core_axis_name="c", subcore_axis_name="s"),
    compiler_params=pltpu.CompilerParams(needs_layout_passes=False),
    scratch_types=[
        pltpu.VMEM((_BPW,), jnp.int32),
        pltpu.VMEM((_BPW,), jnp.int32),
        pltpu.VMEM((_G, DIM, _CH), jnp.float32),
        pltpu.VMEM((_G, DIM, _CH), jnp.float32),
        pltpu.VMEM((_BPW,), jnp.float32),
        pltpu.SemaphoreType.DMA,
        pltpu.SemaphoreType.DMA,
    ],
)


def kernel(idx1, idx2, emb1, emb2, table1, table2):
    del emb1, emb2
    return _sc_reduce(idx1, idx2, table1.T, table2.T)
